# baseline (speedup 1.0000x reference)
import functools
import jax
import jax.numpy as jnp
from jax import lax
from jax.experimental import pallas as pl
from jax.experimental.pallas import tpu as pltpu
from jax.experimental.pallas import tpu_sc as plsc

_K = 2
_TT = 4096


def _tc_body(x_ref, w_ref, b_ref, t1v, t2v, e0, e1, psum, scnt, zsum):
    tt, e = x_ref.shape[1], w_ref.shape[1]
    ti = pl.program_id(1)
    x = x_ref[0]
    logits = jnp.dot(x, w_ref[...], preferred_element_type=jnp.float32)
    logits = logits + b_ref[0][None, :]
    lt = logits.T
    m = jnp.max(lt, axis=0, keepdims=True)
    u = jnp.exp(lt - m)
    s = jnp.sum(u, axis=0, keepdims=True)
    probs = u / s
    logz = m + jnp.log(s)
    row = lax.broadcasted_iota(jnp.int32, (e, tt), 0)
    v1 = jnp.max(probs, axis=0, keepdims=True)
    i1 = jnp.min(jnp.where(probs == v1, row, e), axis=0, keepdims=True)
    h1 = row == i1
    pm = jnp.where(h1, -1.0, probs)
    v2 = jnp.max(pm, axis=0, keepdims=True)
    i2 = jnp.min(jnp.where(pm == v2, row, e), axis=0, keepdims=True)
    h2 = row == i2
    t1v[...] = v1[None]
    t2v[...] = v2[None]
    e0[...] = i1[None]
    e1[...] = i2[None]

    ps = jnp.sum(probs, axis=1, keepdims=True)[None]
    sc = jnp.sum((h1 | h2).astype(jnp.float32), axis=1, keepdims=True)[None]
    zs = jnp.sum(logz * logz).reshape(1, 1, 1)

    @pl.when(ti == 0)
    def _():
        psum[...] = jnp.zeros_like(psum)
        scnt[...] = jnp.zeros_like(scnt)
        zsum[...] = jnp.zeros_like(zsum)

    psum[...] += ps
    scnt[...] += sc
    zsum[...] += zs


def _tc_stage(token_inputs, W, b):
    G, T, H = token_inputs.shape
    E = W.shape[1]
    NT = T // _TT
    return pl.pallas_call(
        _tc_body,
        grid=(G, NT),
        in_specs=[
            pl.BlockSpec((1, _TT, H), lambda g, t: (g, t, 0)),
            pl.BlockSpec((H, E), lambda g, t: (0, 0)),
            pl.BlockSpec((1, E), lambda g, t: (0, 0)),
        ],
        out_specs=[
            pl.BlockSpec((1, 1, _TT), lambda g, t: (g * (T // _TT) + t, 0, 0)),
            pl.BlockSpec((1, 1, _TT), lambda g, t: (g * (T // _TT) + t, 0, 0)),
            pl.BlockSpec((1, 1, _TT), lambda g, t: (g * (T // _TT) + t, 0, 0)),
            pl.BlockSpec((1, 1, _TT), lambda g, t: (g * (T // _TT) + t, 0, 0)),
            pl.BlockSpec((1, E, 1), lambda g, t: (g, 0, 0)),
            pl.BlockSpec((1, E, 1), lambda g, t: (g, 0, 0)),
            pl.BlockSpec((1, 1, 1), lambda g, t: (g, 0, 0)),
        ],
        out_shape=[
            jax.ShapeDtypeStruct((G * NT, 1, _TT), jnp.float32),
            jax.ShapeDtypeStruct((G * NT, 1, _TT), jnp.float32),
            jax.ShapeDtypeStruct((G * NT, 1, _TT), jnp.int32),
            jax.ShapeDtypeStruct((G * NT, 1, _TT), jnp.int32),
            jax.ShapeDtypeStruct((G, E, 1), jnp.float32),
            jax.ShapeDtypeStruct((G, E, 1), jnp.float32),
            jax.ShapeDtypeStruct((G, 1, 1), jnp.float32),
        ],
    )(token_inputs, W, b.reshape(1, E))


def _dup_ranks(b, base):
    cf = plsc.scan_count(b)[0] - base
    cb = lax.rev(plsc.scan_count(lax.rev(b, (0,)))[0], (0,)) - base
    return cf, cf + cb + 1


def _sc_route(T, E, cap):
    CH = 1024
    NVC = CH // 16
    NCH = T // CH
    QW = T

    def body(t1v_hbm, t2v_hbm, e0_hbm, e1_hbm, cap_hbm,
             disp_out, comb_out,
             t1c, t2c, e0f, e1f, key_c, tok_c,
             histl, hall, offs, posb, val0, val1,
             ehl, ehall, cnt0, cnt1,
             pri0c, pri1c, disp_c, comb_c, cap_v, esem0, esem1,
             sh_keyA, sh_tokA, sh_keyB, sh_tokB,
             sh_hist, sh_eh, sh_pri0, sh_pri1):
        cid = lax.axis_index("c")
        sid = lax.axis_index("s")
        q = sid // 8
        ch = sid % 8
        g = cid * 2 + q
        qo = q * QW
        co = ch * CH

        pltpu.sync_copy(t1v_hbm.at[g, pl.ds(co, CH)], t1c)
        e0cp = pltpu.async_copy(e0_hbm.at[g], e0f, esem0)
        e1cp = pltpu.async_copy(e1_hbm.at[g], e1f, esem1)
        pltpu.sync_copy(cap_hbm, cap_v)

        zeros16 = jnp.zeros((16,), jnp.int32)
        ones16 = jnp.ones((16,), jnp.int32)
        base = jnp.min(plsc.scan_count(zeros16)[0])

        def _sl(i):
            return pl.ds(pl.multiple_of(i * 16, 16), 16)

        @plsc.parallel_loop(0, NVC, unroll=4)
        def init_body(i):
            bits = plsc.bitcast(t1c[_sl(i)], jnp.int32)
            ik = 0x3F800000 - bits
            ik = jnp.minimum(jnp.maximum(ik, 0), 0x03FFFFFF)
            key_c[_sl(i)] = ik
            tok_c[_sl(i)] = lax.iota(jnp.int32, 16) + (co + i * 16)

        def radix_pass(shift, src_k_sh, src_t_sh, dst_k_sh, dst_t_sh):
            if src_k_sh is not None:
                pltpu.sync_copy(src_k_sh.at[pl.ds(qo + co, CH)], key_c)
                pltpu.sync_copy(src_t_sh.at[pl.ds(qo + co, CH)], tok_c)

            @plsc.parallel_loop(0, 32, unroll=4)
            def hz(i):
                histl[_sl(i)] = zeros16

            @pl.loop(0, NVC, unroll=4)
            def hb(i):
                b = (key_c[_sl(i)] >> shift) & 511
                plsc.addupdate_scatter(histl, [b], ones16)

            pltpu.sync_copy(histl, sh_hist.at[pl.ds(q * (NCH * 512) + ch * 512, 512)])
            plsc.subcore_barrier()
            pltpu.sync_copy(sh_hist.at[pl.ds(q * (NCH * 512), NCH * 512)], hall)

            def pf(j, carry):
                tot = zeros16
                myp = zeros16
                for c2 in range(NCH):
                    v = hall[pl.ds(pl.multiple_of(c2 * 512 + j * 16, 16), 16)]
                    tot = tot + v
                    myp = myp + jnp.where(c2 < ch, v, 0)
                inc = plsc.cumsum(tot) + carry
                offs[_sl(j)] = inc - tot + myp
                return jnp.max(inc)
            lax.fori_loop(0, 32, pf, jnp.int32(0))

            @pl.loop(0, NVC, unroll=4)
            def sb(i):
                b = (key_c[_sl(i)] >> shift) & 511
                rank = plsc.scan_count(b)[0] - base
                go = plsc.load_gather(offs, [b])
                posb[i // 8, pl.ds(pl.multiple_of((i % 8) * 16, 16), 16)] = go + rank + qo
                plsc.addupdate_scatter(offs, [b], ones16)

            for j in range(8):
                pltpu.sync_copy(key_c.at[pl.ds(j * 128, 128)], dst_k_sh.at[posb.at[j]])
                pltpu.sync_copy(tok_c.at[pl.ds(j * 128, 128)], dst_t_sh.at[posb.at[j]])
            plsc.subcore_barrier()

        radix_pass(0, None, None, sh_keyA, sh_tokA)
        radix_pass(9, sh_keyA, sh_tokA, sh_keyB, sh_tokB)
        radix_pass(18, sh_keyB, sh_tokB, sh_keyA, sh_tokA)

        pltpu.sync_copy(sh_tokA.at[pl.ds(qo + co, CH)], tok_c)
        e0cp.wait()
        e1cp.wait()

        @plsc.parallel_loop(0, 8, unroll=4)
        def ehz(i):
            ehl[_sl(i)] = zeros16

        @pl.loop(0, NVC, unroll=4)
        def ehb(i):
            t = tok_c[_sl(i)]
            ee0 = plsc.load_gather(e0f, [t])
            ee1 = plsc.load_gather(e1f, [t])
            plsc.addupdate_scatter(ehl, [ee0], ones16)
            plsc.addupdate_scatter(ehl, [ee1 + E], ones16)

        pltpu.sync_copy(ehl, sh_eh.at[pl.ds(q * (NCH * 2 * E) + ch * (2 * E), 2 * E)])
        plsc.subcore_barrier()
        pltpu.sync_copy(sh_eh.at[pl.ds(q * (NCH * 2 * E), NCH * 2 * E)], ehall)

        for j in range(E // 16):
            t0 = zeros16
            t1_ = zeros16
            m0 = zeros16
            m1 = zeros16
            for c2 in range(NCH):
                v0 = ehall[pl.ds(c2 * 2 * E + j * 16, 16)]
                v1 = ehall[pl.ds(c2 * 2 * E + E + j * 16, 16)]
                t0 = t0 + v0
                t1_ = t1_ + v1
                m0 = m0 + jnp.where(c2 < ch, v0, 0)
                m1 = m1 + jnp.where(c2 < ch, v1, 0)
            cnt0[pl.ds(j * 16, 16)] = m0
            cnt1[pl.ds(j * 16, 16)] = t0 + m1

        @pl.loop(0, NVC, unroll=4)
        def ck0(i):
            t = tok_c[_sl(i)]
            ee = plsc.load_gather(e0f, [t])
            rank = plsc.scan_count(ee)[0] - base
            gc = plsc.load_gather(cnt0, [ee])
            val0[_sl(i)] = gc + rank
            plsc.addupdate_scatter(cnt0, [ee], ones16)
            posb[i // 8, pl.ds(pl.multiple_of((i % 8) * 16, 16), 16)] = t + qo

        @pl.loop(0, NVC, unroll=4)
        def ck1(i):
            t = tok_c[_sl(i)]
            ee = plsc.load_gather(e1f, [t])
            rank = plsc.scan_count(ee)[0] - base
            gc = plsc.load_gather(cnt1, [ee])
            val1[_sl(i)] = gc + rank
            plsc.addupdate_scatter(cnt1, [ee], ones16)

        for j in range(8):
            pltpu.sync_copy(val0.at[pl.ds(j * 128, 128)], sh_pri0.at[posb.at[j]])
            pltpu.sync_copy(val1.at[pl.ds(j * 128, 128)], sh_pri1.at[posb.at[j]])
        plsc.subcore_barrier()

        pltpu.sync_copy(sh_pri0.at[pl.ds(qo + co, CH)], pri0c)
        pltpu.sync_copy(sh_pri1.at[pl.ds(qo + co, CH)], pri1c)
        pltpu.sync_copy(t2v_hbm.at[g, pl.ds(co, CH)], t2c)
        capv_ = cap_v[...]

        @plsc.parallel_loop(0, NVC, unroll=2)
        def ob(i):
            il = lax.iota(jnp.int32, 16) + i * 16
            p0 = pri0c[_sl(i)]
            p1 = pri1c[_sl(i)]
            ee0 = e0f[pl.ds(pl.multiple_of(co + i * 16, 16), 16)]
            ee1 = e1f[pl.ds(pl.multiple_of(co + i * 16, 16), 16)]
            c0 = jnp.where(p0 < capv_, t1c[_sl(i)], 0.0)
            c1 = jnp.where(p1 < capv_, t2c[_sl(i)], 0.0)
            i4 = il * 4
            i2 = il * 2
            plsc.store_scatter(disp_c, [i4], ee0)
            plsc.store_scatter(disp_c, [i4 + 1], p0)
            plsc.store_scatter(disp_c, [i4 + 2], ee1)
            plsc.store_scatter(disp_c, [i4 + 3], p1)
            plsc.store_scatter(comb_c, [i2], c0)
            plsc.store_scatter(comb_c, [i2 + 1], c1)

        pltpu.sync_copy(disp_c, disp_out.at[g, pl.ds(co * 4, CH * 4)])
        pltpu.sync_copy(comb_c, comb_out.at[g, pl.ds(co * 2, CH * 2)])

    G = 4
    mesh = plsc.VectorSubcoreMesh(core_axis_name="c", subcore_axis_name="s")
    return pl.kernel(
        body,
        mesh=mesh,
        compiler_params=pltpu.CompilerParams(needs_layout_passes=False),
        out_type=[
            jax.ShapeDtypeStruct((G, T * 4), jnp.int32),
            jax.ShapeDtypeStruct((G, T * 2), jnp.float32),
        ],
        scratch_types=[
            pltpu.VMEM((CH,), jnp.float32),
            pltpu.VMEM((CH,), jnp.float32),
            pltpu.VMEM((T,), jnp.int32),
            pltpu.VMEM((T,), jnp.int32),
            pltpu.VMEM((CH,), jnp.int32),
            pltpu.VMEM((CH,), jnp.int32),
            pltpu.VMEM((512,), jnp.int32),
            pltpu.VMEM((T // CH * 512,), jnp.int32),
            pltpu.VMEM((512,), jnp.int32),
            pltpu.VMEM((8, 128), jnp.int32),
            pltpu.VMEM((CH,), jnp.int32),
            pltpu.VMEM((CH,), jnp.int32),
            pltpu.VMEM((2 * E,), jnp.int32),
            pltpu.VMEM((T // CH * 2 * E,), jnp.int32),
            pltpu.VMEM((E,), jnp.int32),
            pltpu.VMEM((E,), jnp.int32),
            pltpu.VMEM((CH,), jnp.int32),
            pltpu.VMEM((CH,), jnp.int32),
            pltpu.VMEM((CH * 4,), jnp.int32),
            pltpu.VMEM((CH * 2,), jnp.float32),
            pltpu.VMEM((16,), jnp.int32),
            pltpu.SemaphoreType.DMA,
            pltpu.SemaphoreType.DMA,
            pltpu.VMEM_SHARED((2 * T,), jnp.int32),
            pltpu.VMEM_SHARED((2 * T,), jnp.int32),
            pltpu.VMEM_SHARED((2 * T,), jnp.int32),
            pltpu.VMEM_SHARED((2 * T,), jnp.int32),
            pltpu.VMEM_SHARED((2 * (T // CH) * 512,), jnp.int32),
            pltpu.VMEM_SHARED((2 * (T // CH) * 2 * E,), jnp.int32),
            pltpu.VMEM_SHARED((2 * T,), jnp.int32),
            pltpu.VMEM_SHARED((2 * T,), jnp.int32),
        ],
    )


def kernel(token_inputs, num_experts, expert_capacity, W, b):
    token_inputs = token_inputs.astype(jnp.float32)
    G, T, H = token_inputs.shape
    E = W.shape[1]

    t1v, t2v, e0, e1, psum, scnt, zsum = _tc_stage(token_inputs, W, b)
    t1v = t1v.reshape(G, T)
    t2v = t2v.reshape(G, T)
    e0 = e0.reshape(G, T)
    e1 = e1.reshape(G, T)
    psum = psum.reshape(G, E)
    scnt = scnt.reshape(G, E)

    cap_arr = jnp.full((16,), expert_capacity, jnp.int32)
    dispatch, combine = _sc_route(T, E, None)(t1v, t2v, e0, e1, cap_arr)
    dispatch = dispatch.reshape(G, T, 2, 2)
    combine = combine.reshape(G, T, 2)

    aux = jnp.sum(scnt * psum) * (float(E * E) / (float(G * E) * float(T) * float(T)))
    z = jnp.sum(zsum) / (G * T)
    return dispatch.astype(jnp.int32), combine.astype(jnp.float32), aux, z


# speedup vs baseline: 1.4226x; 1.4226x over previous
import functools
import jax
import jax.numpy as jnp
from jax import lax
from jax.experimental import pallas as pl
from jax.experimental.pallas import tpu as pltpu
from jax.experimental.pallas import tpu_sc as plsc

_K = 2
_TT = 8192


def _tc_body(x_ref, w_ref, b_ref, t1v, t2v, e0, e1, psum, scnt, zsum):
    tt, e = x_ref.shape[1], w_ref.shape[1]
    ti = pl.program_id(1)
    x = x_ref[0]
    logits = jnp.dot(x, w_ref[...], preferred_element_type=jnp.float32)
    logits = logits + b_ref[0][None, :]
    lt = logits.T
    m = jnp.max(lt, axis=0, keepdims=True)
    u = jnp.exp(lt - m)
    s = jnp.sum(u, axis=0, keepdims=True)
    probs = u / s
    logz = m + jnp.log(s)
    row = lax.broadcasted_iota(jnp.int32, (e, tt), 0)
    v1 = jnp.max(probs, axis=0, keepdims=True)
    i1 = jnp.min(jnp.where(probs == v1, row, e), axis=0, keepdims=True)
    h1 = row == i1
    pm = jnp.where(h1, -1.0, probs)
    v2 = jnp.max(pm, axis=0, keepdims=True)
    i2 = jnp.min(jnp.where(pm == v2, row, e), axis=0, keepdims=True)
    h2 = row == i2
    t1v[...] = v1[None]
    t2v[...] = v2[None]
    e0[...] = i1[None]
    e1[...] = i2[None]

    ps = jnp.sum(probs, axis=1, keepdims=True)[None]
    sc = jnp.sum((h1 | h2).astype(jnp.float32), axis=1, keepdims=True)[None]
    zs = jnp.sum(logz * logz).reshape(1, 1, 1)

    @pl.when(ti == 0)
    def _():
        psum[...] = jnp.zeros_like(psum)
        scnt[...] = jnp.zeros_like(scnt)
        zsum[...] = jnp.zeros_like(zsum)

    psum[...] += ps
    scnt[...] += sc
    zsum[...] += zs


def _tc_stage(token_inputs, W, b):
    G, T, H = token_inputs.shape
    E = W.shape[1]
    NT = T // _TT
    return pl.pallas_call(
        _tc_body,
        grid=(G, NT),
        in_specs=[
            pl.BlockSpec((1, _TT, H), lambda g, t: (g, t, 0)),
            pl.BlockSpec((H, E), lambda g, t: (0, 0)),
            pl.BlockSpec((1, E), lambda g, t: (0, 0)),
        ],
        out_specs=[
            pl.BlockSpec((1, 1, _TT), lambda g, t: (g * (T // _TT) + t, 0, 0)),
            pl.BlockSpec((1, 1, _TT), lambda g, t: (g * (T // _TT) + t, 0, 0)),
            pl.BlockSpec((1, 1, _TT), lambda g, t: (g * (T // _TT) + t, 0, 0)),
            pl.BlockSpec((1, 1, _TT), lambda g, t: (g * (T // _TT) + t, 0, 0)),
            pl.BlockSpec((1, E, 1), lambda g, t: (g, 0, 0)),
            pl.BlockSpec((1, E, 1), lambda g, t: (g, 0, 0)),
            pl.BlockSpec((1, 1, 1), lambda g, t: (g, 0, 0)),
        ],
        out_shape=[
            jax.ShapeDtypeStruct((G * NT, 1, _TT), jnp.float32),
            jax.ShapeDtypeStruct((G * NT, 1, _TT), jnp.float32),
            jax.ShapeDtypeStruct((G * NT, 1, _TT), jnp.int32),
            jax.ShapeDtypeStruct((G * NT, 1, _TT), jnp.int32),
            jax.ShapeDtypeStruct((G, E, 1), jnp.float32),
            jax.ShapeDtypeStruct((G, E, 1), jnp.float32),
            jax.ShapeDtypeStruct((G, 1, 1), jnp.float32),
        ],
    )(token_inputs, W, b.reshape(1, E))


def _dup_ranks(b, base):
    cf = plsc.scan_count(b)[0] - base
    cb = lax.rev(plsc.scan_count(lax.rev(b, (0,)))[0], (0,)) - base
    return cf, cf + cb + 1


def _sc_route(T, E, cap):
    CH = 1024
    NVC = CH // 16
    NCH = T // CH
    QW = T

    def body(t1v_hbm, t2v_hbm, e0_hbm, e1_hbm, cap_hbm,
             pri0_out, pri1_out, c0_out, c1_out,
             t1c, t2c, e0f, e1f, key_c, tok_c,
             histl, hall, offs, posb, val0, val1,
             ehl, ehall, cnt0, cnt1,
             pri0c, pri1c, c0c, c1c, cap_v, esem0, esem1,
             sh_keyA, sh_tokA, sh_keyB, sh_tokB,
             sh_hist, sh_eh, sh_pri0, sh_pri1):
        cid = lax.axis_index("c")
        sid = lax.axis_index("s")
        q = sid // 8
        ch = sid % 8
        g = cid * 2 + q
        qo = q * QW
        co = ch * CH

        pltpu.sync_copy(t1v_hbm.at[g, pl.ds(co, CH)], t1c)
        e0cp = pltpu.async_copy(e0_hbm.at[g], e0f, esem0)
        e1cp = pltpu.async_copy(e1_hbm.at[g], e1f, esem1)
        pltpu.sync_copy(cap_hbm, cap_v)

        zeros16 = jnp.zeros((16,), jnp.int32)
        ones16 = jnp.ones((16,), jnp.int32)
        base = jnp.min(plsc.scan_count(zeros16)[0])

        def _sl(i):
            return pl.ds(pl.multiple_of(i * 16, 16), 16)

        @plsc.parallel_loop(0, NVC, unroll=4)
        def init_body(i):
            bits = plsc.bitcast(t1c[_sl(i)], jnp.int32)
            ik = 0x3F800000 - bits
            ik = jnp.minimum(jnp.maximum(ik, 0), 0x03FFFFFF)
            key_c[_sl(i)] = ik
            tok_c[_sl(i)] = lax.iota(jnp.int32, 16) + (co + i * 16)

        def radix_pass(shift, src_k_sh, src_t_sh, dst_k_sh, dst_t_sh):
            if src_k_sh is not None:
                pltpu.sync_copy(src_k_sh.at[pl.ds(qo + co, CH)], key_c)
                pltpu.sync_copy(src_t_sh.at[pl.ds(qo + co, CH)], tok_c)

            @plsc.parallel_loop(0, 32, unroll=4)
            def hz(i):
                histl[_sl(i)] = zeros16

            @pl.loop(0, NVC, unroll=4)
            def hb(i):
                b = (key_c[_sl(i)] >> shift) & 511
                plsc.addupdate_scatter(histl, [b], ones16)

            pltpu.sync_copy(histl, sh_hist.at[pl.ds(q * (NCH * 512) + ch * 512, 512)])
            plsc.subcore_barrier()
            pltpu.sync_copy(sh_hist.at[pl.ds(q * (NCH * 512), NCH * 512)], hall)

            def pf(j, carry):
                tot = zeros16
                myp = zeros16
                for c2 in range(NCH):
                    v = hall[pl.ds(pl.multiple_of(c2 * 512 + j * 16, 16), 16)]
                    tot = tot + v
                    myp = myp + jnp.where(c2 < ch, v, 0)
                inc = plsc.cumsum(tot) + carry
                offs[_sl(j)] = inc - tot + myp
                return jnp.max(inc)
            lax.fori_loop(0, 32, pf, jnp.int32(0))

            @pl.loop(0, NVC, unroll=4)
            def sb(i):
                b = (key_c[_sl(i)] >> shift) & 511
                rank = plsc.scan_count(b)[0] - base
                go = plsc.load_gather(offs, [b])
                posb[i // 8, pl.ds(pl.multiple_of((i % 8) * 16, 16), 16)] = go + rank + qo
                plsc.addupdate_scatter(offs, [b], ones16)

            for j in range(8):
                pltpu.sync_copy(key_c.at[pl.ds(j * 128, 128)], dst_k_sh.at[posb.at[j]])
                pltpu.sync_copy(tok_c.at[pl.ds(j * 128, 128)], dst_t_sh.at[posb.at[j]])
            plsc.subcore_barrier()

        radix_pass(0, None, None, sh_keyA, sh_tokA)
        radix_pass(9, sh_keyA, sh_tokA, sh_keyB, sh_tokB)
        radix_pass(18, sh_keyB, sh_tokB, sh_keyA, sh_tokA)

        pltpu.sync_copy(sh_tokA.at[pl.ds(qo + co, CH)], tok_c)
        e0cp.wait()
        e1cp.wait()

        @plsc.parallel_loop(0, 8, unroll=4)
        def ehz(i):
            ehl[_sl(i)] = zeros16

        @pl.loop(0, NVC, unroll=4)
        def ehb(i):
            t = tok_c[_sl(i)]
            ee0 = plsc.load_gather(e0f, [t])
            ee1 = plsc.load_gather(e1f, [t])
            plsc.addupdate_scatter(ehl, [ee0], ones16)
            plsc.addupdate_scatter(ehl, [ee1 + E], ones16)

        pltpu.sync_copy(ehl, sh_eh.at[pl.ds(q * (NCH * 2 * E) + ch * (2 * E), 2 * E)])
        plsc.subcore_barrier()
        pltpu.sync_copy(sh_eh.at[pl.ds(q * (NCH * 2 * E), NCH * 2 * E)], ehall)

        for j in range(E // 16):
            t0 = zeros16
            t1_ = zeros16
            m0 = zeros16
            m1 = zeros16
            for c2 in range(NCH):
                v0 = ehall[pl.ds(c2 * 2 * E + j * 16, 16)]
                v1 = ehall[pl.ds(c2 * 2 * E + E + j * 16, 16)]
                t0 = t0 + v0
                t1_ = t1_ + v1
                m0 = m0 + jnp.where(c2 < ch, v0, 0)
                m1 = m1 + jnp.where(c2 < ch, v1, 0)
            cnt0[pl.ds(j * 16, 16)] = m0
            cnt1[pl.ds(j * 16, 16)] = t0 + m1

        @pl.loop(0, NVC, unroll=4)
        def ck0(i):
            t = tok_c[_sl(i)]
            ee = plsc.load_gather(e0f, [t])
            rank = plsc.scan_count(ee)[0] - base
            gc = plsc.load_gather(cnt0, [ee])
            val0[_sl(i)] = gc + rank
            plsc.addupdate_scatter(cnt0, [ee], ones16)
            posb[i // 8, pl.ds(pl.multiple_of((i % 8) * 16, 16), 16)] = t + qo

        @pl.loop(0, NVC, unroll=4)
        def ck1(i):
            t = tok_c[_sl(i)]
            ee = plsc.load_gather(e1f, [t])
            rank = plsc.scan_count(ee)[0] - base
            gc = plsc.load_gather(cnt1, [ee])
            val1[_sl(i)] = gc + rank
            plsc.addupdate_scatter(cnt1, [ee], ones16)

        for j in range(8):
            pltpu.sync_copy(val0.at[pl.ds(j * 128, 128)], sh_pri0.at[posb.at[j]])
            pltpu.sync_copy(val1.at[pl.ds(j * 128, 128)], sh_pri1.at[posb.at[j]])
        plsc.subcore_barrier()

        pltpu.sync_copy(sh_pri0.at[pl.ds(qo + co, CH)], pri0c)
        pltpu.sync_copy(sh_pri1.at[pl.ds(qo + co, CH)], pri1c)
        pltpu.sync_copy(t2v_hbm.at[g, pl.ds(co, CH)], t2c)
        capv_ = cap_v[...]

        @plsc.parallel_loop(0, NVC, unroll=4)
        def ob(i):
            p0 = pri0c[_sl(i)]
            p1 = pri1c[_sl(i)]
            c0c[_sl(i)] = jnp.where(p0 < capv_, t1c[_sl(i)], 0.0)
            c1c[_sl(i)] = jnp.where(p1 < capv_, t2c[_sl(i)], 0.0)

        pltpu.sync_copy(pri0c, pri0_out.at[g, pl.ds(co, CH)])
        pltpu.sync_copy(pri1c, pri1_out.at[g, pl.ds(co, CH)])
        pltpu.sync_copy(c0c, c0_out.at[g, pl.ds(co, CH)])
        pltpu.sync_copy(c1c, c1_out.at[g, pl.ds(co, CH)])

    G = 4
    mesh = plsc.VectorSubcoreMesh(core_axis_name="c", subcore_axis_name="s")
    return pl.kernel(
        body,
        mesh=mesh,
        compiler_params=pltpu.CompilerParams(needs_layout_passes=False),
        out_type=[
            jax.ShapeDtypeStruct((G, T), jnp.int32),
            jax.ShapeDtypeStruct((G, T), jnp.int32),
            jax.ShapeDtypeStruct((G, T), jnp.float32),
            jax.ShapeDtypeStruct((G, T), jnp.float32),
        ],
        scratch_types=[
            pltpu.VMEM((CH,), jnp.float32),
            pltpu.VMEM((CH,), jnp.float32),
            pltpu.VMEM((T,), jnp.int32),
            pltpu.VMEM((T,), jnp.int32),
            pltpu.VMEM((CH,), jnp.int32),
            pltpu.VMEM((CH,), jnp.int32),
            pltpu.VMEM((512,), jnp.int32),
            pltpu.VMEM((T // CH * 512,), jnp.int32),
            pltpu.VMEM((512,), jnp.int32),
            pltpu.VMEM((8, 128), jnp.int32),
            pltpu.VMEM((CH,), jnp.int32),
            pltpu.VMEM((CH,), jnp.int32),
            pltpu.VMEM((2 * E,), jnp.int32),
            pltpu.VMEM((T // CH * 2 * E,), jnp.int32),
            pltpu.VMEM((E,), jnp.int32),
            pltpu.VMEM((E,), jnp.int32),
            pltpu.VMEM((CH,), jnp.int32),
            pltpu.VMEM((CH,), jnp.int32),
            pltpu.VMEM((CH,), jnp.float32),
            pltpu.VMEM((CH,), jnp.float32),
            pltpu.VMEM((16,), jnp.int32),
            pltpu.SemaphoreType.DMA,
            pltpu.SemaphoreType.DMA,
            pltpu.VMEM_SHARED((2 * T,), jnp.int32),
            pltpu.VMEM_SHARED((2 * T,), jnp.int32),
            pltpu.VMEM_SHARED((2 * T,), jnp.int32),
            pltpu.VMEM_SHARED((2 * T,), jnp.int32),
            pltpu.VMEM_SHARED((2 * (T // CH) * 512,), jnp.int32),
            pltpu.VMEM_SHARED((2 * (T // CH) * 2 * E,), jnp.int32),
            pltpu.VMEM_SHARED((2 * T,), jnp.int32),
            pltpu.VMEM_SHARED((2 * T,), jnp.int32),
        ],
    )


def kernel(token_inputs, num_experts, expert_capacity, W, b):
    token_inputs = token_inputs.astype(jnp.float32)
    G, T, H = token_inputs.shape
    E = W.shape[1]

    t1v, t2v, e0, e1, psum, scnt, zsum = _tc_stage(token_inputs, W, b)
    t1v = t1v.reshape(G, T)
    t2v = t2v.reshape(G, T)
    e0 = e0.reshape(G, T)
    e1 = e1.reshape(G, T)
    psum = psum.reshape(G, E)
    scnt = scnt.reshape(G, E)

    cap_arr = jnp.full((16,), expert_capacity, jnp.int32)
    pri0, pri1, c0, c1 = _sc_route(T, E, None)(t1v, t2v, e0, e1, cap_arr)

    dispatch = jnp.stack(
        [jnp.stack([e0, pri0], axis=-1), jnp.stack([e1, pri1], axis=-1)],
        axis=-2)
    combine = jnp.stack([c0, c1], axis=-1)

    aux = jnp.sum(scnt * psum) * (float(E * E) / (float(G * E) * float(T) * float(T)))
    z = jnp.sum(zsum) / (G * T)
    return dispatch.astype(jnp.int32), combine.astype(jnp.float32), aux, z
